# baseline (speedup 1.0000x reference)
import functools

import jax
import jax.numpy as jnp
from jax import lax
from jax.experimental import pallas as pl
from jax.experimental.pallas import tpu as pltpu
from jax.experimental.pallas import tpu_sc as plsc

D_MODEL = 64
DPAD = 128
SCALE = 8.0
LANES = 16
IB = 256
NBUF = 2
GAHEAD = 1


@functools.lru_cache(maxsize=None)
def _build(S, N):
    info = plsc.get_sparse_core_info()
    NC, NS = info.num_cores, info.num_subcores
    NW = NC * NS
    nblk = N // IB
    t_per_w = S * nblk // NW
    assert t_per_w % NBUF == 0
    mesh = plsc.VectorSubcoreMesh(core_axis_name="c", subcore_axis_name="s")

    @functools.partial(
        pl.kernel,
        mesh=mesh,
        out_type=jax.ShapeDtypeStruct((S * D_MODEL, N), jnp.float32),
        scratch_types=(
            [pltpu.VMEM((IB,), jnp.int32) for _ in range(NBUF)]
            + [pltpu.VMEM((IB, DPAD), jnp.float32) for _ in range(NBUF)]
            + [pltpu.VMEM((D_MODEL, IB), jnp.float32) for _ in range(NBUF)]
            + [pltpu.SemaphoreType.DMA] * 3
        ),
        compiler_params=pltpu.CompilerParams(needs_layout_passes=False),
    )
    def k(xt_hbm, tp_hbm, out_hbm, *refs):
        idx_v = refs[0:NBUF]
        rows_v = refs[NBUF:2 * NBUF]
        rt_v = refs[2 * NBUF:3 * NBUF]
        isem, gsem, wsem = refs[3 * NBUF:]
        wid = lax.axis_index("s") * NC + lax.axis_index("c")
        t0 = wid * t_per_w

        def task_jc(g):
            t = t0 + g
            return t // nblk, (t % nblk) * IB

        def fire_idx(g, b):
            j, i0 = task_jc(g)
            pltpu.async_copy(xt_hbm.at[j, pl.ds(i0, IB)], idx_v[b], isem)

        def fire_gather(b):
            for h in range(IB // 128):
                pltpu.async_copy(
                    tp_hbm.at[idx_v[b].at[pl.ds(h * 128, 128)]],
                    rows_v[b].at[pl.ds(h * 128, 128)], gsem)

        def drain_rows(b):
            pltpu.make_async_copy(tp_hbm.at[pl.ds(0, IB)], rows_v[b],
                                  gsem).wait()

        def drain_wr(b):
            pltpu.make_async_copy(
                out_hbm.at[pl.ds(0, D_MODEL), pl.ds(0, IB)], rt_v[b],
                wsem).wait()

        def drain_idx(b):
            pltpu.make_async_copy(xt_hbm.at[0, pl.ds(0, IB)], idx_v[b],
                                  isem).wait()

        for b in range(NBUF):
            fire_idx(b, b)
        for b in range(GAHEAD):
            drain_idx(b)
            fire_gather(b)

        cvecs = [lax.iota(jnp.int32, LANES) + (cb * LANES)
                 for cb in range(D_MODEL // LANES)]

        def outer(g2, carry):
            for b in range(NBUF):
                g = g2 * NBUF + b
                drain_rows(b)

                @pl.when(g + NBUF < t_per_w)
                def _():
                    fire_idx(g + NBUF, b)

                @pl.when(g + GAHEAD < t_per_w)
                def _():
                    gb = (b + GAHEAD) % NBUF
                    drain_idx(gb)
                    fire_gather(gb)

                @pl.when(g >= NBUF)
                def _():
                    drain_wr(b)

                rows_b = rows_v[b]
                rt_b = rt_v[b]

                @plsc.parallel_loop(0, IB, unroll=8)
                def tr_body(i):
                    ivec = jnp.full((LANES,), i, jnp.int32)
                    for cb in range(D_MODEL // LANES):
                        v = rows_b[i, pl.ds(cb * LANES, LANES)]
                        plsc.store_scatter(rt_b, [cvecs[cb], ivec],
                                           v * SCALE)

                j, i0 = task_jc(g)
                pltpu.async_copy(
                    rt_b,
                    out_hbm.at[pl.ds(j * D_MODEL, D_MODEL), pl.ds(i0, IB)],
                    wsem,
                )
            return carry

        lax.fori_loop(0, t_per_w // NBUF, outer, 0)
        for b in range(NBUF):
            drain_wr(b)

    return k


def kernel(x, table):
    N, S = x.shape
    xt = jnp.transpose(x).astype(jnp.int32)
    tp = jnp.pad(table, ((0, 0), (0, DPAD - D_MODEL)))
    out2 = _build(S, N)(xt, tp)
    out3 = out2.reshape(S, D_MODEL, N)
    return jnp.transpose(out3, (2, 0, 1))


# speedup vs baseline: 1.1861x; 1.1861x over previous
import functools

import jax
import jax.numpy as jnp
from jax import lax
from jax.experimental import pallas as pl
from jax.experimental.pallas import tpu as pltpu
from jax.experimental.pallas import tpu_sc as plsc

D_MODEL = 64
DPAD = 128
SCALE = 8.0
LANES = 16
IB = 256
NBUF = 2
GAHEAD = 1


@functools.lru_cache(maxsize=None)
def _build(S, N):
    info = plsc.get_sparse_core_info()
    NC, NS = info.num_cores, info.num_subcores
    NW = NC * NS
    nblk = N // IB
    t_per_w = S * nblk // NW
    assert t_per_w % NBUF == 0
    mesh = plsc.VectorSubcoreMesh(core_axis_name="c", subcore_axis_name="s")

    @functools.partial(
        pl.kernel,
        mesh=mesh,
        out_type=jax.ShapeDtypeStruct((S * D_MODEL, N), jnp.float32),
        scratch_types=(
            [pltpu.VMEM((IB,), jnp.int32) for _ in range(NBUF)]
            + [pltpu.VMEM((IB, DPAD), jnp.float32) for _ in range(NBUF)]
            + [pltpu.VMEM((D_MODEL, IB), jnp.float32) for _ in range(NBUF)]
            + [pltpu.SemaphoreType.DMA] * 3
        ),
        compiler_params=pltpu.CompilerParams(needs_layout_passes=False),
    )
    def k(xt_hbm, tp_hbm, out_hbm, *refs):
        idx_v = refs[0:NBUF]
        rows_v = refs[NBUF:2 * NBUF]
        rt_v = refs[2 * NBUF:3 * NBUF]
        isem, gsem, wsem = refs[3 * NBUF:]
        wid = lax.axis_index("s") * NC + lax.axis_index("c")
        t0 = wid * t_per_w

        def task_jc(g):
            t = t0 + g
            return t // nblk, (t % nblk) * IB

        def fire_idx(g, b):
            j, i0 = task_jc(g)
            pltpu.async_copy(xt_hbm.at[j, pl.ds(i0, IB)], idx_v[b], isem)

        def fire_gather(b):
            for h in range(IB // 128):
                pltpu.async_copy(
                    tp_hbm.at[idx_v[b].at[pl.ds(h * 128, 128)]],
                    rows_v[b].at[pl.ds(h * 128, 128)], gsem)

        def drain_rows(b):
            pltpu.make_async_copy(tp_hbm.at[pl.ds(0, IB)], rows_v[b],
                                  gsem).wait()

        def drain_wr(b):
            pltpu.make_async_copy(
                out_hbm.at[pl.ds(0, D_MODEL), pl.ds(0, IB)], rt_v[b],
                wsem).wait()

        def drain_idx(b):
            pltpu.make_async_copy(xt_hbm.at[0, pl.ds(0, IB)], idx_v[b],
                                  isem).wait()

        for b in range(NBUF):
            fire_idx(b, b)
        for b in range(GAHEAD):
            drain_idx(b)
            fire_gather(b)

        cvecs = [lax.iota(jnp.int32, LANES) + (cb * LANES)
                 for cb in range(D_MODEL // LANES)]

        def outer(g2, carry):
            for b in range(NBUF):
                g = g2 * NBUF + b
                drain_rows(b)

                @pl.when(g + NBUF < t_per_w)
                def _():
                    fire_idx(g + NBUF, b)

                @pl.when(g + GAHEAD < t_per_w)
                def _():
                    gb = (b + GAHEAD) % NBUF
                    drain_idx(gb)
                    fire_gather(gb)

                @pl.when(g >= NBUF)
                def _():
                    drain_wr(b)

                rows_b = rows_v[b]
                rt_b = rt_v[b]

                @plsc.parallel_loop(0, IB, unroll=8)
                def tr_body(i):
                    ivec = jnp.full((LANES,), i, jnp.int32)
                    for cb in range(D_MODEL // LANES):
                        v = rows_b[i, pl.ds(cb * LANES, LANES)]
                        plsc.store_scatter(rt_b, [cvecs[cb], ivec],
                                           v * SCALE)

                j, i0 = task_jc(g)
                pltpu.async_copy(
                    rt_b,
                    out_hbm.at[pl.ds(j * D_MODEL, D_MODEL), pl.ds(i0, IB)],
                    wsem,
                )
            return carry

        lax.fori_loop(0, t_per_w // NBUF, outer, 0)
        for b in range(NBUF):
            drain_wr(b)

    return k


BLKV = 4096


@functools.lru_cache(maxsize=None)
def _tc_prep(V):
    def body(in_ref, out_ref):
        out_ref[:, 0:D_MODEL] = jnp.swapaxes(in_ref[...], 0, 1)

    return pl.pallas_call(
        body,
        grid=((V + BLKV - 1) // BLKV,),
        in_specs=[pl.BlockSpec((D_MODEL, BLKV), lambda b: (0, b))],
        out_specs=pl.BlockSpec((BLKV, DPAD), lambda b: (b, 0)),
        out_shape=jax.ShapeDtypeStruct((V, DPAD), jnp.float32),
    )


def kernel(x, table):
    N, S = x.shape
    V = table.shape[0]
    xt = jnp.transpose(x).astype(jnp.int32)
    tp = _tc_prep(V)(jnp.transpose(table))
    out2 = _build(S, N)(xt, tp)
    out3 = out2.reshape(S, D_MODEL, N)
    return jnp.transpose(out3, (2, 0, 1))


# speedup vs baseline: 1.1956x; 1.0080x over previous
import functools

import jax
import jax.numpy as jnp
from jax import lax
from jax.experimental import pallas as pl
from jax.experimental.pallas import tpu as pltpu
from jax.experimental.pallas import tpu_sc as plsc

D_MODEL = 64
DPAD = 128
SCALE = 8.0
LANES = 16
IB = 128
NBUF = 2
GAHEAD = 1


@functools.lru_cache(maxsize=None)
def _build(S, N):
    info = plsc.get_sparse_core_info()
    NC, NS = info.num_cores, info.num_subcores
    NW = NC * NS
    nblk = N // IB
    t_per_w = S * nblk // NW
    assert t_per_w % NBUF == 0
    mesh = plsc.VectorSubcoreMesh(core_axis_name="c", subcore_axis_name="s")

    @functools.partial(
        pl.kernel,
        mesh=mesh,
        out_type=jax.ShapeDtypeStruct((S * D_MODEL, N), jnp.float32),
        scratch_types=(
            [pltpu.VMEM((IB,), jnp.int32) for _ in range(NBUF)]
            + [pltpu.VMEM((IB, DPAD), jnp.float32) for _ in range(NBUF)]
            + [pltpu.VMEM((D_MODEL, IB), jnp.float32) for _ in range(NBUF)]
            + [pltpu.SemaphoreType.DMA] * 3
        ),
        compiler_params=pltpu.CompilerParams(needs_layout_passes=False),
    )
    def k(xt_hbm, tp_hbm, out_hbm, *refs):
        idx_v = refs[0:NBUF]
        rows_v = refs[NBUF:2 * NBUF]
        rt_v = refs[2 * NBUF:3 * NBUF]
        isem, gsem, wsem = refs[3 * NBUF:]
        wid = lax.axis_index("s") * NC + lax.axis_index("c")
        t0 = wid * t_per_w

        def task_jc(g):
            t = t0 + g
            return t // nblk, (t % nblk) * IB

        def fire_idx(g, b):
            j, i0 = task_jc(g)
            pltpu.async_copy(xt_hbm.at[j, pl.ds(i0, IB)], idx_v[b], isem)

        def fire_gather(b):
            for h in range(IB // 128):
                pltpu.async_copy(
                    tp_hbm.at[idx_v[b].at[pl.ds(h * 128, 128)]],
                    rows_v[b].at[pl.ds(h * 128, 128)], gsem)

        def drain_rows(b):
            pltpu.make_async_copy(tp_hbm.at[pl.ds(0, IB)], rows_v[b],
                                  gsem).wait()

        def drain_wr(b):
            pltpu.make_async_copy(
                out_hbm.at[pl.ds(0, D_MODEL), pl.ds(0, IB)], rt_v[b],
                wsem).wait()

        def drain_idx(b):
            pltpu.make_async_copy(xt_hbm.at[0, pl.ds(0, IB)], idx_v[b],
                                  isem).wait()

        for b in range(NBUF):
            fire_idx(b, b)
        for b in range(GAHEAD):
            drain_idx(b)
            fire_gather(b)

        cvecs = [lax.iota(jnp.int32, LANES) + (cb * LANES)
                 for cb in range(D_MODEL // LANES)]

        def outer(g2, carry):
            for b in range(NBUF):
                g = g2 * NBUF + b
                drain_rows(b)

                @pl.when(g + NBUF < t_per_w)
                def _():
                    fire_idx(g + NBUF, b)

                @pl.when(g + GAHEAD < t_per_w)
                def _():
                    gb = (b + GAHEAD) % NBUF
                    drain_idx(gb)
                    fire_gather(gb)

                @pl.when(g >= NBUF)
                def _():
                    drain_wr(b)

                rows_b = rows_v[b]
                rt_b = rt_v[b]

                @plsc.parallel_loop(0, IB, unroll=8)
                def tr_body(i):
                    ivec = jnp.full((LANES,), i, jnp.int32)
                    for cb in range(D_MODEL // LANES):
                        v = rows_b[i, pl.ds(cb * LANES, LANES)]
                        plsc.store_scatter(rt_b, [cvecs[cb], ivec],
                                           v * SCALE)

                j, i0 = task_jc(g)
                pltpu.async_copy(
                    rt_b,
                    out_hbm.at[pl.ds(j * D_MODEL, D_MODEL), pl.ds(i0, IB)],
                    wsem,
                )
            return carry

        lax.fori_loop(0, t_per_w // NBUF, outer, 0)
        for b in range(NBUF):
            drain_wr(b)

    return k


BLKV = 4096


@functools.lru_cache(maxsize=None)
def _tc_prep(V):
    def body(in_ref, out_ref):
        out_ref[:, 0:D_MODEL] = jnp.swapaxes(in_ref[...], 0, 1)

    return pl.pallas_call(
        body,
        grid=((V + BLKV - 1) // BLKV,),
        in_specs=[pl.BlockSpec((D_MODEL, BLKV), lambda b: (0, b))],
        out_specs=pl.BlockSpec((BLKV, DPAD), lambda b: (b, 0)),
        out_shape=jax.ShapeDtypeStruct((V, DPAD), jnp.float32),
    )


def kernel(x, table):
    N, S = x.shape
    V = table.shape[0]
    xt = jnp.transpose(x).astype(jnp.int32)
    tp = _tc_prep(V)(jnp.transpose(table))
    out2 = _build(S, N)(xt, tp)
    out3 = out2.reshape(S, D_MODEL, N)
    return jnp.transpose(out3, (2, 0, 1))
